# baseline (speedup 1.0000x reference)
import functools

import jax
import jax.numpy as jnp
from jax import lax
from jax.experimental import pallas as pl
from jax.experimental.pallas import tpu as pltpu
from jax.experimental.pallas import tpu_sc as plsc

HI = lax.Precision.HIGHEST

N = 10000
NPAD = 10240
RT = 256
NT = NPAD // RT
EDGES = 320000
EPAD = 327680
PER_W = EPAD // 32
CH = 64
NCHUNK = PER_W // CH
DUMMY = N
NG = 100
F32 = jnp.float32

HALF = NPAD // 2
TRASH = HALF


def _sc_messages_body(T_hbm, E_hbm, f_hbm, t_hbm, z_hbm, out_hbm,
                      bufA, bufB, aggsh, sem):
    cid = lax.axis_index("c")
    sid = lax.axis_index("s")

    def zero_upper(buf):
        Fb, Rb = buf[7], buf[8]

        @pl.loop(0, CH)
        def zrow(e):
            zz = jnp.zeros((16,), F32)
            for j in range(4):
                Fb[e, pl.ds(64 + 16 * j, 16)] = zz
                Rb[e, pl.ds(64 + 16 * j, 16)] = zz

    zero_upper(bufA)
    zero_upper(bufB)
    wid = sid * 2 + cid
    base = wid * PER_W

    def do_chunk(off, half, buf):
        fidx, tidx, fs, ts, Tf, Tt, Ec, Fb, Rb = buf
        pltpu.sync_copy(f_hbm.at[pl.ds(off, CH)], fidx)
        pltpu.sync_copy(t_hbm.at[pl.ds(off, CH)], tidx)
        pltpu.async_copy(T_hbm.at[fidx], Tf, sem).wait()
        pltpu.async_copy(T_hbm.at[tidx], Tt, sem).wait()
        pltpu.sync_copy(E_hbm.at[pl.ds(off, CH)], Ec)

        lo = half * HALF

        @pl.loop(0, CH // 16)
        def sidx(k):
            fv = fidx[pl.ds(k * 16, 16)] - lo
            tv = tidx[pl.ds(k * 16, 16)] - lo
            okf = (fv >= 0) & (fv < HALF)
            okt = (tv >= 0) & (tv < HALF)
            fs[pl.ds(k * 16, 16)] = jnp.where(okf, fv, TRASH)
            ts[pl.ds(k * 16, 16)] = jnp.where(okt, tv, TRASH)

        @pl.loop(0, CH)
        def erow(e):
            for j in range(4):
                s = 16 * j
                a = Tf[e, pl.ds(s, 16)]
                b = Tt[e, pl.ds(64 + s, 16)]
                ec = Ec[e, pl.ds(s, 16)]
                Fb[e, pl.ds(s, 16)] = jnp.maximum(a + b + ec, 0.0)
                a2 = Tt[e, pl.ds(s, 16)]
                b2 = Tf[e, pl.ds(64 + s, 16)]
                Rb[e, pl.ds(s, 16)] = jnp.maximum(a2 + b2 + ec, 0.0)

        pltpu.sync_copy(Fb, aggsh.at[ts], add=True)
        pltpu.sync_copy(Rb, aggsh.at[fs], add=True)

    for half in range(2):
        @pl.when(sid == 0)
        def _():
            pltpu.sync_copy(z_hbm, aggsh)

        plsc.subcore_barrier()

        @pl.loop(0, NCHUNK // 2)
        def chunk2(ii):
            do_chunk(base + (ii * 2) * CH, half, bufA)
            do_chunk(base + (ii * 2 + 1) * CH, half, bufB)

        plsc.subcore_barrier()

        @pl.when((sid == 0) & (cid == 0))
        def _():
            pltpu.sync_copy(aggsh.at[pl.ds(0, HALF)],
                            out_hbm.at[0].at[pl.ds(half * HALF, HALF)])

        @pl.when((sid == 0) & (cid == 1))
        def _():
            pltpu.sync_copy(aggsh.at[pl.ds(0, HALF)],
                            out_hbm.at[1].at[pl.ds(half * HALF, HALF)])

        plsc.subcore_barrier()


def _sc_degrees_body(f_hbm, t_hbm, z_hbm, out_hbm, fidx, tidx, ones, degsh):
    cid = lax.axis_index("c")
    sid = lax.axis_index("s")

    @pl.when(sid == 0)
    def _():
        pltpu.sync_copy(z_hbm, degsh)

    @pl.loop(0, CH)
    def fill(e):
        ones[e, pl.ds(0, 16)] = jnp.full((16,), 1.0, F32)

    plsc.subcore_barrier()
    wid = sid * 2 + cid
    base = wid * PER_W

    @pl.loop(0, NCHUNK)
    def chunk(i):
        off = base + i * CH
        pltpu.sync_copy(f_hbm.at[pl.ds(off, CH)], fidx)
        pltpu.sync_copy(t_hbm.at[pl.ds(off, CH)], tidx)
        pltpu.sync_copy(ones, degsh.at[tidx], add=True)
        pltpu.sync_copy(ones, degsh.at[fidx], add=True)

    plsc.subcore_barrier()

    @pl.when((sid == 0) & (cid == 0))
    def _():
        pltpu.sync_copy(degsh, out_hbm.at[0])

    @pl.when((sid == 0) & (cid == 1))
    def _():
        pltpu.sync_copy(degsh, out_hbm.at[1])


@functools.cache
def _sc_kernels():
    mesh = plsc.VectorSubcoreMesh(core_axis_name="c", subcore_axis_name="s")
    msgs = pl.kernel(
        _sc_messages_body,
        mesh=mesh,
        out_type=jax.ShapeDtypeStruct((2, NPAD, 128), F32),
        scratch_types=[
            [pltpu.VMEM((CH,), jnp.int32), pltpu.VMEM((CH,), jnp.int32),
             pltpu.VMEM((CH,), jnp.int32), pltpu.VMEM((CH,), jnp.int32),
             pltpu.VMEM((CH, 128), F32), pltpu.VMEM((CH, 128), F32),
             pltpu.VMEM((CH, 64), F32), pltpu.VMEM((CH, 128), F32),
             pltpu.VMEM((CH, 128), F32)],
            [pltpu.VMEM((CH,), jnp.int32), pltpu.VMEM((CH,), jnp.int32),
             pltpu.VMEM((CH,), jnp.int32), pltpu.VMEM((CH,), jnp.int32),
             pltpu.VMEM((CH, 128), F32), pltpu.VMEM((CH, 128), F32),
             pltpu.VMEM((CH, 64), F32), pltpu.VMEM((CH, 128), F32),
             pltpu.VMEM((CH, 128), F32)],
            pltpu.VMEM_SHARED((NPAD // 2 + 8, 128), F32),
            pltpu.SemaphoreType.DMA,
        ],
    )
    degs = pl.kernel(
        _sc_degrees_body,
        mesh=mesh,
        out_type=jax.ShapeDtypeStruct((2, NPAD, 16), F32),
        scratch_types=[
            pltpu.VMEM((CH,), jnp.int32),
            pltpu.VMEM((CH,), jnp.int32),
            pltpu.VMEM((CH, 16), F32),
            pltpu.VMEM_SHARED((NPAD, 16), F32),
        ],
    )
    return msgs, degs


_BISECT_MSG_JNP = False
_BISECT_DEG_JNP = False


def _sc_messages(T, E, fp, tp, z64):
    if _BISECT_MSG_JNP:
        Tf = T[fp]; Tt = T[tp]
        Fm = jnp.maximum(Tf[:, :64] + Tt[:, 64:] + E, 0.0)
        Rm = jnp.maximum(Tt[:, :64] + Tf[:, 64:] + E, 0.0)
        agg = (jax.ops.segment_sum(Fm, tp, num_segments=NPAD)
               + jax.ops.segment_sum(Rm, fp, num_segments=NPAD))
        agg = jnp.pad(agg, ((0, 0), (0, 64)))
        return jnp.stack([agg, jnp.zeros_like(agg)])
    return _sc_kernels()[0](T, E, fp, tp, z64)


def _sc_degrees(fp, tp, z16):
    if _BISECT_DEG_JNP:
        ones = jnp.ones((EPAD, 16), F32)
        deg = (jax.ops.segment_sum(ones, tp, num_segments=NPAD)
               + jax.ops.segment_sum(ones, fp, num_segments=NPAD))
        return jnp.stack([deg, jnp.zeros_like(deg)])
    return _sc_kernels()[1](fp, tp, z16)


def _enc_edge_body(ef_ref, ev_ref, We_ref, be_ref, out_ref):
    out_ref[...] = (jnp.dot(ef_ref[...], We_ref[...], preferred_element_type=F32, precision=HI)
                    + be_ref[...] * ev_ref[...])


def _enc_node_body(nf_ref, v_ref, W_ref, b_ref, W1_ref, h_ref, t_ref):
    h = jnp.dot(nf_ref[...], W_ref[...], preferred_element_type=F32, precision=HI) + b_ref[...]
    h = jnp.where(v_ref[...] > 0, h, 0.0)
    h_ref[...] = h
    t_ref[...] = jnp.dot(h, W1_ref[...], preferred_element_type=F32, precision=HI)


def _layer_body(lohi, hfull_ref, hblk_ref, gr_ref, pr_ref, gc_ref, pc_ref,
                agg_ref, deg_ref, v_ref, Wm2_ref, bm2_ref, WihT_ref, bih_ref,
                WhhT_ref, bhh_ref, W1_ref, hout_ref, tout_ref):
    r = pl.program_id(0)
    Xr = hblk_ref[...]
    agg2 = agg_ref[0][:, :64] + agg_ref[1][:, :64]
    deg = deg_ref[0][:, :1] + deg_ref[1][:, :1]
    aggm = (jnp.dot(agg2, Wm2_ref[...], preferred_element_type=F32, precision=HI)
            + deg * bm2_ref[...])

    grt = gr_ref[...]
    prt = pr_ref[...]
    m0 = jnp.full((RT, 1), -1e30, F32)
    l0 = jnp.zeros((RT, 1), F32)
    a0 = jnp.zeros((RT, 32), F32)

    def ct_body(ct, carry):
        m, l, acc = carry
        c0 = ct * RT
        Xc = hfull_ref[pl.ds(c0, RT), :]
        gcc = gc_ref[:, pl.ds(c0, RT)]
        pcc = pc_ref[:, pl.ds(c0, RT)]
        S = lax.dot_general(Xr, Xc, (((1,), (1,)), ((), ())),
                            preferred_element_type=F32, precision=HI)
        msk = (prt == pcc) & (grt != gcc)
        S = jnp.where(msk, S, -1e30)
        mnew = jnp.maximum(m, jnp.max(S, axis=1, keepdims=True))
        p = jnp.where(msk, jnp.exp(S - mnew), 0.0)
        alpha = jnp.exp(m - mnew)
        lnew = l * alpha + jnp.sum(p, axis=1, keepdims=True)
        accnew = acc * alpha + jnp.dot(p, Xc, preferred_element_type=F32, precision=HI)
        return mnew, lnew, accnew

    lo = lohi[r]
    hi = lohi[NT + r]
    m, l, acc = lax.fori_loop(lo, hi, ct_body, (m0, l0, a0))
    att = Xr - acc / l

    gx = jnp.concatenate([aggm, att], axis=1)
    gi = jnp.dot(gx, WihT_ref[...], preferred_element_type=F32, precision=HI) + bih_ref[...]
    gh = jnp.dot(Xr, WhhT_ref[...], preferred_element_type=F32, precision=HI) + bhh_ref[...]
    rg = jax.nn.sigmoid(gi[:, :32] + gh[:, :32])
    zg = jax.nn.sigmoid(gi[:, 32:64] + gh[:, 32:64])
    nst = jnp.tanh(gi[:, 64:96] + rg * gh[:, 64:96])
    hnew = (1.0 - zg) * nst + zg * Xr
    hnew = jnp.where(v_ref[...] > 0, hnew, 0.0)
    hout_ref[...] = hnew
    tout_ref[...] = jnp.dot(hnew, W1_ref[...], preferred_element_type=F32, precision=HI)


def _readout_body(h_ref, gcb_ref, Wa1_ref, ba1_ref, out_ref):
    r = pl.program_id(0)

    @pl.when(r == 0)
    def _():
        out_ref[...] = jnp.zeros_like(out_ref)

    g = jnp.dot(h_ref[...], Wa1_ref[...], preferred_element_type=F32, precision=HI) + ba1_ref[...]
    gated = jax.nn.sigmoid(g[:, :128]) * g[:, 128:]
    iota = lax.broadcasted_iota(jnp.int32, (NG, RT), 0).astype(F32)
    ohT = jnp.where(iota == gcb_ref[...], 1.0, 0.0)
    out_ref[...] += jnp.dot(ohT, gated, preferred_element_type=F32, precision=HI)


def _head_body(gs_ref, Se_ref, So_ref, Wa2_ref, ba2_ref, W1a_ref, W1b_ref,
               bf1_ref, Wf2_ref, bf2_ref, out_ref):
    gs = jnp.dot(gs_ref[...], Wa2_ref[...], preferred_element_type=F32, precision=HI) + ba2_ref[...]
    a = jnp.dot(Se_ref[...], gs, preferred_element_type=F32, precision=HI)
    b = jnp.dot(So_ref[...], gs, preferred_element_type=F32, precision=HI)
    hh = jnp.maximum(jnp.dot(a, W1a_ref[...], preferred_element_type=F32, precision=HI)
                     + jnp.dot(b, W1b_ref[...], preferred_element_type=F32, precision=HI)
                     + bf1_ref[...], 0.0)
    out_ref[...] = jnp.dot(hh, Wf2_ref[...], preferred_element_type=F32, precision=HI) + bf2_ref[...]


def _full(shape):
    return pl.BlockSpec(shape, lambda *_: tuple(0 for _ in shape))


def kernel(node_features, edge_features, from_idx, to_idx, graph_idx,
           training_n_graphs_in_batch, W_enc_n, b_enc_n, W_enc_e, b_enc_e,
           W_m1, b_m1, W_m2, b_m2, W_ih, b_ih, W_hh, b_hh, W_a1, b_a1,
           W_a2, b_a2, W_f1, b_f1, W_f2, b_f2):
    n_graphs = training_n_graphs_in_batch

    W1cat = jnp.concatenate([W_m1[:32], W_m1[32:64]], axis=1)
    We_e = jnp.dot(W_enc_e, W_m1[64:80], precision=HI)
    be_e = (jnp.dot(b_enc_e, W_m1[64:80], precision=HI) + b_m1).reshape(1, 64)
    WihT = W_ih.T
    WhhT = W_hh.T
    bih = b_ih.reshape(1, 96)
    bhh = b_hh.reshape(1, 96)
    bm2 = b_m2.reshape(1, 64)
    ba1 = b_a1.reshape(1, 256)
    ba2 = b_a2.reshape(1, 128)
    bf1 = b_f1.reshape(1, 256)
    bf2 = b_f2.reshape(1, 1)
    Wf1a = W_f1[:128]
    Wf1b = W_f1[128:]
    half = jnp.arange(50)
    Se = (half[:, None] * 2 == jnp.arange(NG)[None, :]).astype(F32)
    So = (half[:, None] * 2 + 1 == jnp.arange(NG)[None, :]).astype(F32)

    gid = graph_idx.astype(jnp.int32)
    fp = jnp.concatenate([from_idx.astype(jnp.int32),
                          jnp.full((EPAD - EDGES,), DUMMY, jnp.int32)])
    tp = jnp.concatenate([to_idx.astype(jnp.int32),
                          jnp.full((EPAD - EDGES,), DUMMY, jnp.int32)])
    pad_i = jnp.full((NPAD - N,), -1, jnp.int32)
    gid_r = jnp.concatenate([gid, pad_i])
    pair_r = jnp.concatenate([gid // 2, pad_i])
    gc_real = jnp.where(gid < n_graphs, gid, -2)
    pc_real = jnp.where(gid < n_graphs, gid // 2, -2)
    pad_c = jnp.full((NPAD - N,), -2, jnp.int32)
    gr = gid_r.astype(F32).reshape(NPAD, 1)
    pr = pair_r.astype(F32).reshape(NPAD, 1)
    gc = jnp.concatenate([gc_real, pad_c]).astype(F32).reshape(1, NPAD)
    pc = jnp.concatenate([pc_real, pad_c]).astype(F32).reshape(1, NPAD)
    gcb = jnp.concatenate([gid, pad_c]).astype(F32).reshape(1, NPAD)
    valid = (jnp.arange(NPAD) < N).astype(F32).reshape(NPAD, 1)

    pstart = jnp.searchsorted(gid, jnp.arange(0, 102, 2)).astype(jnp.int32)
    rows0 = jnp.arange(NT, dtype=jnp.int32) * RT
    rows1 = jnp.minimum(rows0 + RT - 1, N - 1)
    pf = gid[jnp.minimum(rows0, N - 1)] // 2
    plast = gid[rows1] // 2
    lo_t = pstart[pf] // RT
    hi_t = (pstart[plast + 1] + RT - 1) // RT
    in_range = rows0 < N
    lo_t = jnp.where(in_range, lo_t, 0)
    hi_t = jnp.where(in_range, hi_t, 0)
    lohi = jnp.concatenate([lo_t, hi_t]).astype(jnp.int32)

    nf_pad = jnp.concatenate(
        [node_features, jnp.zeros((NPAD - N, 128), F32)])
    ef_pad = jnp.concatenate(
        [edge_features, jnp.zeros((EPAD - EDGES, 16), F32)])
    evalid = (jnp.arange(EPAD) < EDGES).astype(F32).reshape(EPAD, 1)
    z64 = jnp.zeros((NPAD // 2 + 8, 128), F32)
    z16 = jnp.zeros((NPAD, 16), F32)

    ET = 512
    E = pl.pallas_call(
        _enc_edge_body,
        grid=(EPAD // ET,),
        in_specs=[pl.BlockSpec((ET, 16), lambda i: (i, 0)),
                  pl.BlockSpec((ET, 1), lambda i: (i, 0)),
                  _full((16, 64)), _full((1, 64))],
        out_specs=pl.BlockSpec((ET, 64), lambda i: (i, 0)),
        out_shape=jax.ShapeDtypeStruct((EPAD, 64), F32),
    )(ef_pad, evalid, We_e, be_e)

    h, T = pl.pallas_call(
        _enc_node_body,
        grid=(NT,),
        in_specs=[pl.BlockSpec((RT, 128), lambda i: (i, 0)),
                  pl.BlockSpec((RT, 1), lambda i: (i, 0)),
                  _full((128, 32)), _full((1, 32)), _full((32, 128))],
        out_specs=[pl.BlockSpec((RT, 32), lambda i: (i, 0)),
                   pl.BlockSpec((RT, 128), lambda i: (i, 0))],
        out_shape=[jax.ShapeDtypeStruct((NPAD, 32), F32),
                   jax.ShapeDtypeStruct((NPAD, 128), F32)],
    )(nf_pad, valid, W_enc_n, b_enc_n.reshape(1, 32), W1cat)

    DEG = _sc_degrees(fp, tp, z16)

    layer = pl.pallas_call(
        _layer_body,
        grid_spec=pltpu.PrefetchScalarGridSpec(
            num_scalar_prefetch=1,
            grid=(NT,),
            in_specs=[
                _full((NPAD, 32)),
                pl.BlockSpec((RT, 32), lambda i, s: (i, 0)),
                pl.BlockSpec((RT, 1), lambda i, s: (i, 0)),
                pl.BlockSpec((RT, 1), lambda i, s: (i, 0)),
                _full((1, NPAD)),
                _full((1, NPAD)),
                pl.BlockSpec((2, RT, 128), lambda i, s: (0, i, 0)),
                pl.BlockSpec((2, RT, 16), lambda i, s: (0, i, 0)),
                pl.BlockSpec((RT, 1), lambda i, s: (i, 0)),
                _full((64, 64)), _full((1, 64)),
                _full((96, 96)), _full((1, 96)),
                _full((32, 96)), _full((1, 96)),
                _full((32, 128)),
            ],
            out_specs=[pl.BlockSpec((RT, 32), lambda i, s: (i, 0)),
                       pl.BlockSpec((RT, 128), lambda i, s: (i, 0))],
        ),
        out_shape=[jax.ShapeDtypeStruct((NPAD, 32), F32),
                   jax.ShapeDtypeStruct((NPAD, 128), F32)],
    )

    for _ in range(5):
        AGG = _sc_messages(T, E, fp, tp, z64)
        h, T = layer(lohi, h, h, gr, pr, gc, pc, AGG, DEG, valid,
                     W_m2, bm2, WihT, bih, WhhT, bhh, W1cat)

    gsum = pl.pallas_call(
        _readout_body,
        grid=(NT,),
        in_specs=[pl.BlockSpec((RT, 32), lambda i: (i, 0)),
                  pl.BlockSpec((1, RT), lambda i: (0, i)),
                  _full((32, 256)), _full((1, 256))],
        out_specs=pl.BlockSpec((NG, 128), lambda i: (0, 0)),
        out_shape=jax.ShapeDtypeStruct((NG, 128), F32),
    )(h, gcb, W_a1, ba1)

    out = pl.pallas_call(
        _head_body,
        in_specs=[_full((NG, 128)), _full((50, NG)), _full((50, NG)),
                  _full((128, 128)), _full((1, 128)),
                  _full((128, 256)), _full((128, 256)), _full((1, 256)),
                  _full((256, 1)), _full((1, 1))],
        out_specs=_full((50, 1)),
        out_shape=jax.ShapeDtypeStruct((50, 1), F32),
    )(gsum, Se, So, W_a2, ba2, Wf1a, Wf1b, bf1, W_f2, bf2)

    return jnp.squeeze(out)
